# Initial kernel scaffold; baseline (speedup 1.0000x reference)
import functools

import jax
import jax.numpy as jnp
from jax import lax
from jax.experimental import pallas as pl
from jax.experimental.pallas import tpu as pltpu
from jax.experimental.pallas import tpu_sc as plsc

N_NODES = 10000
N_EDGES = 320000
D = 128
LANES = 16

NUM_CORES = 2
NUM_SUBCORES = 16
NW = NUM_CORES * NUM_SUBCORES

BATCH = 128
N_CHUNKS = N_EDGES // BATCH
MAX_T = -(-N_CHUNKS // NW)
ROWS_PER_TILE = N_NODES // NUM_SUBCORES


def _make_segsum():
  mesh = plsc.VectorSubcoreMesh(
      core_axis_name="c", subcore_axis_name="s",
      num_cores=NUM_CORES, num_subcores=NUM_SUBCORES)

  @functools.partial(
      pl.kernel,
      out_type=(
          jax.ShapeDtypeStruct((N_NODES, D), jnp.float32),
          jax.ShapeDtypeStruct((N_NODES, D), jnp.float32),
      ),
      mesh=mesh,
      scratch_types=[
          pltpu.VMEM((BATCH,), jnp.int32),
          pltpu.VMEM((BATCH,), jnp.int32),
          pltpu.VMEM((BATCH,), jnp.float32),
          pltpu.VMEM((BATCH, D), jnp.float32),
          pltpu.VMEM_SHARED((N_NODES, D), jnp.float32),
          pltpu.SemaphoreType.DMA,
      ],
  )
  def segsum(h_hbm, src_hbm, dst_hbm, attn_hbm, out0, out1,
             src_v, dst_v, attn_v, rows_v, hn_sh, sem):
    cid = lax.axis_index("c")
    sid = lax.axis_index("s")
    wid = cid * NUM_SUBCORES + sid

    zero = jnp.zeros((LANES,), jnp.float32)

    def zrow(i, carry):
      for dcol in range(D // LANES):
        rows_v[i, pl.ds(dcol * LANES, LANES)] = zero
      return carry

    lax.fori_loop(0, BATCH, zrow, 0)
    for j in range(5):
      pltpu.sync_copy(
          rows_v.at[pl.ds(0, 125)],
          hn_sh.at[pl.ds(sid * ROWS_PER_TILE + j * 125, 125)])
    plsc.subcore_barrier()

    def chunk_body(t, carry):
      chunk = t * NW + wid

      @pl.when(chunk < N_CHUNKS)
      def _():
        base = chunk * BATCH
        pltpu.sync_copy(src_hbm.at[pl.ds(base, BATCH)], src_v)
        pltpu.sync_copy(dst_hbm.at[pl.ds(base, BATCH)], dst_v)
        pltpu.sync_copy(attn_hbm.at[pl.ds(base, BATCH)], attn_v)
        pltpu.async_copy(h_hbm.at[src_v], rows_v, sem).wait()

        def scale_body(e, c2):
          a = attn_v[e]
          for dcol in range(D // LANES):
            sl = pl.ds(dcol * LANES, LANES)
            rows_v[e, sl] = rows_v[e, sl] * a
          return c2

        lax.fori_loop(0, BATCH, scale_body, 0)
        pltpu.sync_copy(rows_v, hn_sh.at[dst_v], add=True)

      return carry

    lax.fori_loop(0, MAX_T, chunk_body, 0)
    plsc.subcore_barrier()

    @pl.when(cid == 0)
    def _():
      pltpu.sync_copy(hn_sh.at[pl.ds(sid * ROWS_PER_TILE, ROWS_PER_TILE)],
                      out0.at[pl.ds(sid * ROWS_PER_TILE, ROWS_PER_TILE)])

    @pl.when(cid == 1)
    def _():
      pltpu.sync_copy(hn_sh.at[pl.ds(sid * ROWS_PER_TILE, ROWS_PER_TILE)],
                      out1.at[pl.ds(sid * ROWS_PER_TILE, ROWS_PER_TILE)])

  return segsum


_segsum = _make_segsum()

_BN = 2000


def _bi_body(h_ref, p0_ref, p1_ref, w1_ref, b1_ref, w2_ref, b2_ref, o_ref):
  h = h_ref[...]
  hn = p0_ref[...] + p1_ref[...]
  s = h + hn
  m = h * hn
  dn = (((1,), (1,)), ((), ()))
  a = lax.dot_general(s, w1_ref[...], dn,
                      preferred_element_type=jnp.float32,
                      precision=lax.Precision.HIGHEST) + b1_ref[...]
  b = lax.dot_general(m, w2_ref[...], dn,
                      preferred_element_type=jnp.float32,
                      precision=lax.Precision.HIGHEST) + b2_ref[...]
  o_ref[...] = (jnp.where(a >= 0, a, 0.01 * a)
                + jnp.where(b >= 0, b, 0.01 * b))


_bi_call = pl.pallas_call(
    _bi_body,
    grid=(N_NODES // _BN,),
    in_specs=[
        pl.BlockSpec((_BN, D), lambda i: (i, 0)),
        pl.BlockSpec((_BN, D), lambda i: (i, 0)),
        pl.BlockSpec((_BN, D), lambda i: (i, 0)),
        pl.BlockSpec((D, D), lambda i: (0, 0)),
        pl.BlockSpec((1, D), lambda i: (0, 0)),
        pl.BlockSpec((D, D), lambda i: (0, 0)),
        pl.BlockSpec((1, D), lambda i: (0, 0)),
    ],
    out_specs=pl.BlockSpec((_BN, D), lambda i: (i, 0)),
    out_shape=jax.ShapeDtypeStruct((N_NODES, D), jnp.float32),
)


def _layer(h, src, dst, attn, W1, b1, W2, b2):
  p0, p1 = _segsum(h, src, dst, attn)
  return _bi_call(h, p0, p1, W1, b1.reshape(1, D), W2, b2.reshape(1, D))


def kernel(x, edge_index, edge_attn, W1_0, b1_0, W2_0, b2_0,
           W1_1, b1_1, W2_1, b2_1):
  src = edge_index[0]
  dst = edge_index[1]
  h1 = _layer(x, src, dst, edge_attn, W1_0, b1_0, W2_0, b2_0)
  h2 = _layer(h1, src, dst, edge_attn, W1_1, b1_1, W2_1, b2_1)
  return jnp.concatenate([x, h1, h2], axis=1)


# speedup vs baseline: 5.0710x; 5.0710x over previous
import functools

import jax
import jax.numpy as jnp
from jax import lax
from jax.experimental import pallas as pl
from jax.experimental.pallas import tpu as pltpu
from jax.experimental.pallas import tpu_sc as plsc

N_NODES = 10000
N_EDGES = 320000
D = 128
LANES = 16

NUM_CORES = 2
NUM_SUBCORES = 16
NW = NUM_CORES * NUM_SUBCORES

BATCH = 128
N_CHUNKS = N_EDGES // BATCH
MAX_T = -(-N_CHUNKS // NW)
STRIPE = 624


def _make_segsum():
  mesh = plsc.VectorSubcoreMesh(
      core_axis_name="c", subcore_axis_name="s",
      num_cores=NUM_CORES, num_subcores=NUM_SUBCORES)

  @functools.partial(
      pl.kernel,
      out_type=(
          jax.ShapeDtypeStruct((N_NODES, D), jnp.float32),
          jax.ShapeDtypeStruct((N_NODES, D), jnp.float32),
      ),
      mesh=mesh,
      scratch_types=[
          pltpu.VMEM((BATCH,), jnp.int32),
          pltpu.VMEM((BATCH,), jnp.int32),
          pltpu.VMEM((BATCH,), jnp.float32),
          pltpu.VMEM((BATCH, D), jnp.float32),
          pltpu.VMEM_SHARED((N_NODES, D), jnp.float32),
          pltpu.SemaphoreType.DMA,
      ],
  )
  def segsum(h_hbm, src_hbm, dst_hbm, attn_hbm, out0, out1,
             src_v, dst_v, attn_v, rows_v, hn_sh, sem):
    cid = lax.axis_index("c")
    sid = lax.axis_index("s")
    wid = cid * NUM_SUBCORES + sid

    zero = jnp.zeros((LANES,), jnp.float32)

    def zrow(i, carry):
      for dcol in range(D // LANES):
        rows_v[i, pl.ds(dcol * LANES, LANES)] = zero
      return carry

    lax.fori_loop(0, BATCH, zrow, 0)
    for off, sz in ((0, 128), (128, 128), (256, 128), (384, 128), (512, 112)):
      pltpu.sync_copy(rows_v.at[pl.ds(0, sz)],
                      hn_sh.at[pl.ds(sid * STRIPE + off, sz)])

    @pl.when(sid == NUM_SUBCORES - 1)
    def _():
      pltpu.sync_copy(rows_v.at[pl.ds(0, 16)],
                      hn_sh.at[pl.ds(NUM_SUBCORES * STRIPE, 16)])

    plsc.subcore_barrier()

    def chunk_body(t, carry):
      chunk = t * NW + wid

      @pl.when(chunk < N_CHUNKS)
      def _():
        base = chunk * BATCH
        pltpu.sync_copy(src_hbm.at[pl.ds(base, BATCH)], src_v)
        pltpu.sync_copy(dst_hbm.at[pl.ds(base, BATCH)], dst_v)
        pltpu.sync_copy(attn_hbm.at[pl.ds(base, BATCH)], attn_v)
        pltpu.async_copy(h_hbm.at[src_v], rows_v, sem).wait()

        def scale_body(g, c2):
          a16 = attn_v[pl.ds(g * LANES, LANES)]
          for i in range(LANES):
            a = a16[i]
            e = g * LANES + i
            for dcol in range(D // LANES):
              sl = pl.ds(dcol * LANES, LANES)
              rows_v[e, sl] = rows_v[e, sl] * a
          return c2

        lax.fori_loop(0, BATCH // LANES, scale_body, 0)
        pltpu.sync_copy(rows_v, hn_sh.at[dst_v], add=True)

      return carry

    lax.fori_loop(0, MAX_T, chunk_body, 0)
    plsc.subcore_barrier()

    for c, out in ((0, out0), (1, out1)):

      @pl.when(cid == c)
      def _(out=out):
        pltpu.sync_copy(hn_sh.at[pl.ds(sid * STRIPE, STRIPE)],
                        out.at[pl.ds(sid * STRIPE, STRIPE)])

        @pl.when(sid == NUM_SUBCORES - 1)
        def _():
          pltpu.sync_copy(hn_sh.at[pl.ds(NUM_SUBCORES * STRIPE, 16)],
                          out.at[pl.ds(NUM_SUBCORES * STRIPE, 16)])

  return segsum


_segsum = _make_segsum()

_BN = 2000


def _bi_body(h_ref, p0_ref, p1_ref, w1_ref, b1_ref, w2_ref, b2_ref, o_ref):
  h = h_ref[...]
  hn = p0_ref[...] + p1_ref[...]
  s = h + hn
  m = h * hn
  dn = (((1,), (1,)), ((), ()))
  a = lax.dot_general(s, w1_ref[...], dn,
                      preferred_element_type=jnp.float32,
                      precision=lax.Precision.HIGHEST) + b1_ref[...]
  b = lax.dot_general(m, w2_ref[...], dn,
                      preferred_element_type=jnp.float32,
                      precision=lax.Precision.HIGHEST) + b2_ref[...]
  o_ref[...] = (jnp.where(a >= 0, a, 0.01 * a)
                + jnp.where(b >= 0, b, 0.01 * b))


_bi_call = pl.pallas_call(
    _bi_body,
    grid=(N_NODES // _BN,),
    in_specs=[
        pl.BlockSpec((_BN, D), lambda i: (i, 0)),
        pl.BlockSpec((_BN, D), lambda i: (i, 0)),
        pl.BlockSpec((_BN, D), lambda i: (i, 0)),
        pl.BlockSpec((D, D), lambda i: (0, 0)),
        pl.BlockSpec((1, D), lambda i: (0, 0)),
        pl.BlockSpec((D, D), lambda i: (0, 0)),
        pl.BlockSpec((1, D), lambda i: (0, 0)),
    ],
    out_specs=pl.BlockSpec((_BN, D), lambda i: (i, 0)),
    out_shape=jax.ShapeDtypeStruct((N_NODES, D), jnp.float32),
)


def _layer(h, src, dst, attn, W1, b1, W2, b2):
  p0, p1 = _segsum(h, src, dst, attn)
  return _bi_call(h, p0, p1, W1, b1.reshape(1, D), W2, b2.reshape(1, D))


def kernel(x, edge_index, edge_attn, W1_0, b1_0, W2_0, b2_0,
           W1_1, b1_1, W2_1, b2_1):
  src = edge_index[0]
  dst = edge_index[1]
  h1 = _layer(x, src, dst, edge_attn, W1_0, b1_0, W2_0, b2_0)
  h2 = _layer(h1, src, dst, edge_attn, W1_1, b1_1, W2_1, b2_1)
  return jnp.concatenate([x, h1, h2], axis=1)


# speedup vs baseline: 6.8368x; 1.3482x over previous
import functools

import jax
import jax.numpy as jnp
from jax import lax
from jax.experimental import pallas as pl
from jax.experimental.pallas import tpu as pltpu
from jax.experimental.pallas import tpu_sc as plsc

N_NODES = 10000
N_EDGES = 320000
D = 128
LANES = 16

NUM_CORES = 2
NUM_SUBCORES = 16
NW = NUM_CORES * NUM_SUBCORES

BATCH = 112
NBUF = 3
IDEPTH = 6
CPW = 90
N_CHUNKS_PAD = CPW * NW
E_PAD = N_CHUNKS_PAD * BATCH
assert E_PAD >= N_EDGES
STRIPE = 624


def _make_segsum():
  mesh = plsc.VectorSubcoreMesh(
      core_axis_name="c", subcore_axis_name="s",
      num_cores=NUM_CORES, num_subcores=NUM_SUBCORES)

  @functools.partial(
      pl.kernel,
      out_type=(
          jax.ShapeDtypeStruct((N_NODES, D), jnp.float32),
          jax.ShapeDtypeStruct((N_NODES, D), jnp.float32),
      ),
      mesh=mesh,
      scratch_types=[
          [pltpu.VMEM((BATCH,), jnp.int32) for _ in range(IDEPTH)],
          [pltpu.VMEM((BATCH,), jnp.int32) for _ in range(IDEPTH)],
          [pltpu.VMEM((BATCH,), jnp.float32) for _ in range(IDEPTH)],
          [pltpu.VMEM((BATCH, D), jnp.float32) for _ in range(NBUF)],
          pltpu.VMEM_SHARED((N_NODES, D), jnp.float32),
          [pltpu.SemaphoreType.DMA for _ in range(IDEPTH)],
          [pltpu.SemaphoreType.DMA for _ in range(NBUF)],
          [pltpu.SemaphoreType.DMA for _ in range(NBUF)],
      ],
  )
  def segsum(h_hbm, src_hbm, dst_hbm, attn_hbm, out0, out1,
             src_i, dst_i, attn_i, rows, hn_sh, i_sem, g_sem, s_sem):
    cid = lax.axis_index("c")
    sid = lax.axis_index("s")
    wid = cid * NUM_SUBCORES + sid
    base = wid * CPW * BATCH

    def issue_idx(t, d):
      off = base + t * BATCH
      pltpu.async_copy(src_hbm.at[pl.ds(off, BATCH)], src_i[d], i_sem[d])
      pltpu.async_copy(dst_hbm.at[pl.ds(off, BATCH)], dst_i[d], i_sem[d])
      pltpu.async_copy(attn_hbm.at[pl.ds(off, BATCH)], attn_i[d], i_sem[d])

    def wait_idx(t, d):
      off = base + t * BATCH
      pltpu.make_async_copy(
          src_hbm.at[pl.ds(off, BATCH)], src_i[d], i_sem[d]).wait()
      pltpu.make_async_copy(
          dst_hbm.at[pl.ds(off, BATCH)], dst_i[d], i_sem[d]).wait()
      pltpu.make_async_copy(
          attn_hbm.at[pl.ds(off, BATCH)], attn_i[d], i_sem[d]).wait()

    for t in range(4):
      issue_idx(t, t)
    for t in range(2):
      wait_idx(t, t)
      pltpu.async_copy(h_hbm.at[src_i[t]], rows[t], g_sem[t])

    zbuf = rows[NBUF - 1]
    zero = jnp.zeros((LANES,), jnp.float32)

    def zrow(i, carry):
      for dcol in range(D // LANES):
        zbuf[i, pl.ds(dcol * LANES, LANES)] = zero
      return carry

    lax.fori_loop(0, BATCH, zrow, 0)
    for off, sz in ((0, 112), (112, 112), (224, 112), (336, 112),
                    (448, 112), (560, 64)):
      pltpu.sync_copy(zbuf.at[pl.ds(0, sz)],
                      hn_sh.at[pl.ds(sid * STRIPE + off, sz)])

    @pl.when(sid == NUM_SUBCORES - 1)
    def _():
      pltpu.sync_copy(zbuf.at[pl.ds(0, 16)],
                      hn_sh.at[pl.ds(NUM_SUBCORES * STRIPE, 16)])

    plsc.subcore_barrier()

    def scale(buf, attn_ref):
      def scale_body(g, c2):
        a16 = attn_ref[pl.ds(g * LANES, LANES)]
        for i in range(LANES):
          a = a16[i]
          for dcol in range(D // LANES):
            sl = pl.ds(dcol * LANES, LANES)
            buf[g * LANES + i, sl] = buf[g * LANES + i, sl] * a
        return c2

      lax.fori_loop(0, BATCH // LANES, scale_body, 0, unroll=2)

    def slot(t, j, d):
      buf = rows[j]
      j2 = (j + 2) % NBUF
      d2 = (d + 2) % IDEPTH
      pltpu.make_async_copy(h_hbm.at[src_i[d]], buf, g_sem[j]).wait()
      scale(buf, attn_i[d])

      @pl.when(t + 2 < CPW)
      def _():
        @pl.when(t >= 1)
        def _():
          pltpu.make_async_copy(
              rows[j2], hn_sh.at[dst_i[(d + IDEPTH - 1) % IDEPTH]],
              s_sem[j2]).wait()

        wait_idx(t + 2, d2)
        pltpu.async_copy(h_hbm.at[src_i[d2]], rows[j2], g_sem[j2])

      pltpu.async_copy(buf, hn_sh.at[dst_i[d]], s_sem[j], add=True)

      @pl.when(t + 4 < CPW)
      def _():
        issue_idx(t + 4, (d + 4) % IDEPTH)

    def pipe_body(k, carry):
      for jj in range(IDEPTH):
        t = k * IDEPTH + jj
        slot(t, jj % NBUF, jj)
      return carry

    lax.fori_loop(0, CPW // IDEPTH, pipe_body, 0)

    for t in range(CPW - NBUF, CPW):
      pltpu.make_async_copy(
          rows[t % NBUF], hn_sh.at[dst_i[t % IDEPTH]], s_sem[t % NBUF]).wait()
    plsc.subcore_barrier()

    for c, out in ((0, out0), (1, out1)):

      @pl.when(cid == c)
      def _(out=out):
        pltpu.sync_copy(hn_sh.at[pl.ds(sid * STRIPE, STRIPE)],
                        out.at[pl.ds(sid * STRIPE, STRIPE)])

        @pl.when(sid == NUM_SUBCORES - 1)
        def _():
          pltpu.sync_copy(hn_sh.at[pl.ds(NUM_SUBCORES * STRIPE, 16)],
                          out.at[pl.ds(NUM_SUBCORES * STRIPE, 16)])

  return segsum


_segsum = _make_segsum()

_BN = 2000


def _bi_body(h_ref, p0_ref, p1_ref, w1_ref, b1_ref, w2_ref, b2_ref, o_ref):
  h = h_ref[...]
  hn = p0_ref[...] + p1_ref[...]
  s = h + hn
  m = h * hn
  dn = (((1,), (1,)), ((), ()))
  a = lax.dot_general(s, w1_ref[...], dn,
                      preferred_element_type=jnp.float32,
                      precision=lax.Precision.HIGHEST) + b1_ref[...]
  b = lax.dot_general(m, w2_ref[...], dn,
                      preferred_element_type=jnp.float32,
                      precision=lax.Precision.HIGHEST) + b2_ref[...]
  o_ref[...] = (jnp.where(a >= 0, a, 0.01 * a)
                + jnp.where(b >= 0, b, 0.01 * b))


_bi_call = pl.pallas_call(
    _bi_body,
    grid=(N_NODES // _BN,),
    in_specs=[
        pl.BlockSpec((_BN, D), lambda i: (i, 0)),
        pl.BlockSpec((_BN, D), lambda i: (i, 0)),
        pl.BlockSpec((_BN, D), lambda i: (i, 0)),
        pl.BlockSpec((D, D), lambda i: (0, 0)),
        pl.BlockSpec((1, D), lambda i: (0, 0)),
        pl.BlockSpec((D, D), lambda i: (0, 0)),
        pl.BlockSpec((1, D), lambda i: (0, 0)),
    ],
    out_specs=pl.BlockSpec((_BN, D), lambda i: (i, 0)),
    out_shape=jax.ShapeDtypeStruct((N_NODES, D), jnp.float32),
)


def _layer(h, src, dst, attn, W1, b1, W2, b2):
  p0, p1 = _segsum(h, src, dst, attn)
  return _bi_call(h, p0, p1, W1, b1.reshape(1, D), W2, b2.reshape(1, D))


def kernel(x, edge_index, edge_attn, W1_0, b1_0, W2_0, b2_0,
           W1_1, b1_1, W2_1, b2_1):
  npad = E_PAD - N_EDGES
  src = jnp.concatenate([edge_index[0], jnp.zeros((npad,), jnp.int32)])
  dst = jnp.concatenate([edge_index[1], jnp.zeros((npad,), jnp.int32)])
  attn = jnp.concatenate([edge_attn, jnp.zeros((npad,), jnp.float32)])
  h1 = _layer(x, src, dst, attn, W1_0, b1_0, W2_0, b2_0)
  h2 = _layer(h1, src, dst, attn, W1_1, b1_1, W2_1, b2_1)
  return jnp.concatenate([x, h1, h2], axis=1)


# speedup vs baseline: 6.8416x; 1.0007x over previous
import functools

import jax
import jax.numpy as jnp
from jax import lax
from jax.experimental import pallas as pl
from jax.experimental.pallas import tpu as pltpu
from jax.experimental.pallas import tpu_sc as plsc

N_NODES = 10000
N_EDGES = 320000
D = 128
LANES = 16

NUM_CORES = 2
NUM_SUBCORES = 16
NW = NUM_CORES * NUM_SUBCORES

BATCH = 112
NBUF = 3
IDEPTH = 6
CPW = 90
N_CHUNKS_PAD = CPW * NW
E_PAD = N_CHUNKS_PAD * BATCH
assert E_PAD >= N_EDGES
STRIPE = 624


def _make_segsum():
  mesh = plsc.VectorSubcoreMesh(
      core_axis_name="c", subcore_axis_name="s",
      num_cores=NUM_CORES, num_subcores=NUM_SUBCORES)

  @functools.partial(
      pl.kernel,
      out_type=(
          jax.ShapeDtypeStruct((N_NODES, D), jnp.float32),
          jax.ShapeDtypeStruct((N_NODES, D), jnp.float32),
      ),
      mesh=mesh,
      scratch_types=[
          [pltpu.VMEM((BATCH,), jnp.int32) for _ in range(IDEPTH)],
          [pltpu.VMEM((BATCH,), jnp.int32) for _ in range(IDEPTH)],
          [pltpu.VMEM((BATCH,), jnp.float32) for _ in range(IDEPTH)],
          [pltpu.VMEM((BATCH, D), jnp.float32) for _ in range(NBUF)],
          pltpu.VMEM_SHARED((N_NODES, D), jnp.float32),
          [pltpu.SemaphoreType.DMA for _ in range(IDEPTH)],
          [pltpu.SemaphoreType.DMA for _ in range(NBUF)],
          [pltpu.SemaphoreType.DMA for _ in range(NBUF)],
      ],
  )
  def segsum(h_hbm, src_hbm, dst_hbm, attn_hbm, out0, out1,
             src_i, dst_i, attn_i, rows, hn_sh, i_sem, g_sem, s_sem):
    cid = lax.axis_index("c")
    sid = lax.axis_index("s")
    wid = cid * NUM_SUBCORES + sid
    base = wid * CPW * BATCH

    def issue_idx(t, d):
      off = base + t * BATCH
      pltpu.async_copy(src_hbm.at[pl.ds(off, BATCH)], src_i[d], i_sem[d])
      pltpu.async_copy(dst_hbm.at[pl.ds(off, BATCH)], dst_i[d], i_sem[d])
      pltpu.async_copy(attn_hbm.at[pl.ds(off, BATCH)], attn_i[d], i_sem[d])

    def wait_idx(t, d):
      off = base + t * BATCH
      pltpu.make_async_copy(
          src_hbm.at[pl.ds(off, BATCH)], src_i[d], i_sem[d]).wait()
      pltpu.make_async_copy(
          dst_hbm.at[pl.ds(off, BATCH)], dst_i[d], i_sem[d]).wait()
      pltpu.make_async_copy(
          attn_hbm.at[pl.ds(off, BATCH)], attn_i[d], i_sem[d]).wait()

    for t in range(4):
      issue_idx(t, t)
    for t in range(2):
      wait_idx(t, t)
      pltpu.async_copy(h_hbm.at[src_i[t]], rows[t], g_sem[t])

    zbuf = rows[NBUF - 1]
    zero = jnp.zeros((LANES,), jnp.float32)

    def zrow(i, carry):
      for dcol in range(D // LANES):
        zbuf[i, pl.ds(dcol * LANES, LANES)] = zero
      return carry

    lax.fori_loop(0, BATCH, zrow, 0)
    for off, sz in ((0, 112), (112, 112), (224, 112), (336, 112),
                    (448, 112), (560, 64)):
      pltpu.sync_copy(zbuf.at[pl.ds(0, sz)],
                      hn_sh.at[pl.ds(sid * STRIPE + off, sz)])

    @pl.when(sid == NUM_SUBCORES - 1)
    def _():
      pltpu.sync_copy(zbuf.at[pl.ds(0, 16)],
                      hn_sh.at[pl.ds(NUM_SUBCORES * STRIPE, 16)])

    plsc.subcore_barrier()

    def scale(buf, attn_ref):
      def scale_body(g, c2):
        a16 = attn_ref[pl.ds(g * LANES, LANES)]
        for i in range(LANES):
          av = lax.gather(
              a16, jnp.full((LANES, 1), i, jnp.int32),
              lax.GatherDimensionNumbers(
                  offset_dims=(), collapsed_slice_dims=(0,),
                  start_index_map=(0,)),
              (1,), mode=lax.GatherScatterMode.PROMISE_IN_BOUNDS)
          for dcol in range(D // LANES):
            sl = pl.ds(dcol * LANES, LANES)
            buf[g * LANES + i, sl] = buf[g * LANES + i, sl] * av
        return c2

      lax.fori_loop(0, BATCH // LANES, scale_body, 0, unroll=2)

    def slot(t, j, d):
      buf = rows[j]
      j2 = (j + 2) % NBUF
      d2 = (d + 2) % IDEPTH
      pltpu.make_async_copy(h_hbm.at[src_i[d]], buf, g_sem[j]).wait()
      scale(buf, attn_i[d])

      @pl.when(t + 2 < CPW)
      def _():
        @pl.when(t >= 1)
        def _():
          pltpu.make_async_copy(
              rows[j2], hn_sh.at[dst_i[(d + IDEPTH - 1) % IDEPTH]],
              s_sem[j2]).wait()

        wait_idx(t + 2, d2)
        pltpu.async_copy(h_hbm.at[src_i[d2]], rows[j2], g_sem[j2])

      pltpu.async_copy(buf, hn_sh.at[dst_i[d]], s_sem[j], add=True)

      @pl.when(t + 4 < CPW)
      def _():
        issue_idx(t + 4, (d + 4) % IDEPTH)

    def pipe_body(k, carry):
      for jj in range(IDEPTH):
        t = k * IDEPTH + jj
        slot(t, jj % NBUF, jj)
      return carry

    lax.fori_loop(0, CPW // IDEPTH, pipe_body, 0)

    for t in range(CPW - NBUF, CPW):
      pltpu.make_async_copy(
          rows[t % NBUF], hn_sh.at[dst_i[t % IDEPTH]], s_sem[t % NBUF]).wait()
    plsc.subcore_barrier()

    for c, out in ((0, out0), (1, out1)):

      @pl.when(cid == c)
      def _(out=out):
        pltpu.sync_copy(hn_sh.at[pl.ds(sid * STRIPE, STRIPE)],
                        out.at[pl.ds(sid * STRIPE, STRIPE)])

        @pl.when(sid == NUM_SUBCORES - 1)
        def _():
          pltpu.sync_copy(hn_sh.at[pl.ds(NUM_SUBCORES * STRIPE, 16)],
                          out.at[pl.ds(NUM_SUBCORES * STRIPE, 16)])

  return segsum


_segsum = _make_segsum()

_BN = 2000


def _bi_body(h_ref, p0_ref, p1_ref, w1_ref, b1_ref, w2_ref, b2_ref, o_ref):
  h = h_ref[...]
  hn = p0_ref[...] + p1_ref[...]
  s = h + hn
  m = h * hn
  dn = (((1,), (1,)), ((), ()))
  a = lax.dot_general(s, w1_ref[...], dn,
                      preferred_element_type=jnp.float32,
                      precision=lax.Precision.HIGHEST) + b1_ref[...]
  b = lax.dot_general(m, w2_ref[...], dn,
                      preferred_element_type=jnp.float32,
                      precision=lax.Precision.HIGHEST) + b2_ref[...]
  o_ref[...] = (jnp.where(a >= 0, a, 0.01 * a)
                + jnp.where(b >= 0, b, 0.01 * b))


_bi_call = pl.pallas_call(
    _bi_body,
    grid=(N_NODES // _BN,),
    in_specs=[
        pl.BlockSpec((_BN, D), lambda i: (i, 0)),
        pl.BlockSpec((_BN, D), lambda i: (i, 0)),
        pl.BlockSpec((_BN, D), lambda i: (i, 0)),
        pl.BlockSpec((D, D), lambda i: (0, 0)),
        pl.BlockSpec((1, D), lambda i: (0, 0)),
        pl.BlockSpec((D, D), lambda i: (0, 0)),
        pl.BlockSpec((1, D), lambda i: (0, 0)),
    ],
    out_specs=pl.BlockSpec((_BN, D), lambda i: (i, 0)),
    out_shape=jax.ShapeDtypeStruct((N_NODES, D), jnp.float32),
)


def _layer(h, src, dst, attn, W1, b1, W2, b2):
  p0, p1 = _segsum(h, src, dst, attn)
  return _bi_call(h, p0, p1, W1, b1.reshape(1, D), W2, b2.reshape(1, D))


def kernel(x, edge_index, edge_attn, W1_0, b1_0, W2_0, b2_0,
           W1_1, b1_1, W2_1, b2_1):
  npad = E_PAD - N_EDGES
  src = jnp.concatenate([edge_index[0], jnp.zeros((npad,), jnp.int32)])
  dst = jnp.concatenate([edge_index[1], jnp.zeros((npad,), jnp.int32)])
  attn = jnp.concatenate([edge_attn, jnp.zeros((npad,), jnp.float32)])
  h1 = _layer(x, src, dst, attn, W1_0, b1_0, W2_0, b2_0)
  h2 = _layer(h1, src, dst, attn, W1_1, b1_1, W2_1, b2_1)
  return jnp.concatenate([x, h1, h2], axis=1)
